# baseline (speedup 1.0000x reference)
import functools
import math

import jax
import jax.numpy as jnp
from jax import lax
from jax.experimental import pallas as pl
from jax.experimental.pallas import tpu as pltpu
from jax.experimental.pallas import tpu_sc as plsc

D = 64
LANES = 16
NCORE = 2
NSUB = 16
NW = NCORE * NSUB

VOCAB = 1000000
BATCH = 4096
SEQ = 200
BBLK = BATCH // NW
VPAIR = VOCAB // 2

NFULL = VOCAB // 128
NTAIL = VOCAB - NFULL * 128
BLK_BASE = NFULL // NW
BLK_REM = NFULL - BLK_BASE * NW

SCALE = math.sqrt(D)

_SC_PARAMS = pltpu.CompilerParams(
    use_tc_tiling_on_sc=True, needs_layout_passes=False,
    disable_bounds_checks=True,
)


def _mesh():
    return plsc.VectorSubcoreMesh(
        core_axis_name="c", subcore_axis_name="s",
        num_cores=NCORE, num_subcores=NSUB,
    )


def _fmt_body(tt_hbm, tail_hbm, tp_hbm, s0, s1, d0, d1, gi0, gi1, go0, go1):
    sblk = (s0, s1)
    dblk = (d0, d1)
    isem = (gi0, gi1)
    osem = (go0, go1)

    c = lax.axis_index("c")
    s = lax.axis_index("s")
    wid = s * NCORE + c
    extra = jnp.minimum(wid, BLK_REM)
    start = wid * BLK_BASE + extra
    nblk = BLK_BASE + jnp.where(wid < BLK_REM, 1, 0)

    iota = lax.iota(jnp.int32, LANES)

    def start_in(i, b):
        vb = start + i
        pltpu.async_copy(
            tt_hbm.at[:, pl.ds(vb * 128, 128)], sblk[b], isem[b]
        )

    def wait_in(b):
        pltpu.make_async_copy(
            tt_hbm.at[:, pl.ds(0, 128)], sblk[b], isem[b]
        ).wait()

    def start_out(i, b):
        vb = start + i
        pltpu.async_copy(
            dblk[b], tp_hbm.at[pl.ds(vb * 64, 64)], osem[b]
        )

    def wait_out(b):
        pltpu.make_async_copy(
            dblk[b], tp_hbm.at[pl.ds(0, 64)], osem[b]
        ).wait()

    def transpose_block(b, nq):
        src = sblk[b]
        dst = dblk[b]

        @pl.loop(0, nq, unroll=4)
        def _q(q):
            for k in range(8):
                h = k // 4
                rvec = iota + ((k * LANES) % D)
                cvec = jnp.broadcast_to(2 * q + h, (LANES,))
                vals = plsc.load_gather(src, [rvec, cvec])
                dst[q, pl.ds(k * LANES, LANES)] = vals

    start_in(0, 0)

    @pl.when(nblk > 1)
    def _():
        start_in(1, 1)

    wait_in(0)
    transpose_block(0, 64)
    start_out(0, 0)

    @pl.loop(1, nblk)
    def _blk(i):
        b = lax.rem(i, 2)

        @pl.when(b == 0)
        def _():
            @pl.when(i + 1 < nblk)
            def _():
                start_in(i + 1, 1)
            wait_in(0)
            wait_out(0)
            transpose_block(0, 64)
            start_out(i, 0)

        @pl.when(b == 1)
        def _():
            @pl.when(i + 1 < nblk)
            def _():
                start_in(i + 1, 0)
            wait_in(1)

            @pl.when(i > 2)
            def _():
                wait_out(1)

            transpose_block(1, 64)
            start_out(i, 1)

    wait_out(0)
    wait_out(1)

    @pl.when(wid == NW - 1)
    def _tail():
        pltpu.sync_copy(tail_hbm, dblk[0].at[pl.ds(0, NTAIL // 2)])
        pltpu.sync_copy(dblk[0].at[pl.ds(0, NTAIL // 2)],
                        tp_hbm.at[pl.ds(NFULL * 64, NTAIL // 2)])


@jax.jit
def _format_table(tt, tail_pairs):
    run = functools.partial(
        pl.kernel,
        out_type=jax.ShapeDtypeStruct((VPAIR, 2 * D), jnp.float32),
        mesh=_mesh(),
        scratch_types=[
            pltpu.VMEM((D, 128), jnp.float32),
            pltpu.VMEM((D, 128), jnp.float32),
            pltpu.VMEM((D, 2 * D), jnp.float32),
            pltpu.VMEM((D, 2 * D), jnp.float32),
            pltpu.SemaphoreType.DMA,
            pltpu.SemaphoreType.DMA,
            pltpu.SemaphoreType.DMA,
            pltpu.SemaphoreType.DMA,
        ],
        compiler_params=_SC_PARAMS,
    )(_fmt_body)
    return run(tt, tail_pairs)


def _gather_body(tp_hbm, idx_hbm, out_hbm,
                 idx_v, p0, p1, h0, h1, r0, r1, o0, o1,
                 g0, g1, q0, q1):
    pidx = (p0, p1)
    hoff = (h0, h1)
    rows = (r0, r1)
    outv = (o0, o1)
    gsem = (g0, g1)
    osem = (q0, q1)

    c = lax.axis_index("c")
    s = lax.axis_index("s")
    wid = s * NCORE + c
    bbase = wid * BBLK

    pltpu.sync_copy(idx_hbm.at[:, pl.ds(bbase, BBLK)], idx_v)

    iota = lax.iota(jnp.int32, LANES)

    def prep_indices(s2, b):
        for j in range(BBLK // LANES):
            sl = pl.ds(j * LANES, LANES)
            v = idx_v[s2, sl]
            pidx[b][sl] = lax.shift_right_logical(v, 1)
            hoff[b][sl] = (v & 1) * D

    def start_gather(b):
        pltpu.async_copy(tp_hbm.at[pidx[b]], rows[b], gsem[b])

    def wait_gather(b):
        pltpu.make_async_copy(tp_hbm.at[pidx[b]], rows[b], gsem[b]).wait()

    def start_store(s_now, b):
        pltpu.async_copy(
            outv[b], out_hbm.at[s_now, :, pl.ds(bbase, BBLK)], osem[b]
        )

    def wait_store(b):
        pltpu.make_async_copy(
            outv[b], out_hbm.at[0, :, pl.ds(bbase, BBLK)], osem[b]
        ).wait()

    def transpose_scale(b):
        src = rows[b]
        dst = outv[b]

        @pl.loop(0, D, unroll=4)
        def _f(f):
            for j in range(BBLK // LANES):
                sl = pl.ds(j * LANES, LANES)
                rvec = iota + (j * LANES)
                cvec = hoff[b][sl] + f
                vals = plsc.load_gather(src, [rvec, cvec])
                dst[f, sl] = vals * SCALE

    def step(s_now, b, *, storewait, gather):
        wait_gather(b)
        if storewait:
            wait_store(b)
        transpose_scale(b)
        start_store(s_now, b)
        if gather:
            prep_indices(s_now + 2, b)
            start_gather(b)

    for b in range(2):
        prep_indices(b, b)
        start_gather(b)

    for s_now in range(2):
        step(s_now, s_now, storewait=False, gather=True)

    @pl.loop(1, SEQ // 2 - 1)
    def _main(i):
        for b in range(2):
            step(i * 2 + b, b, storewait=True, gather=True)

    for s_now in range(SEQ - 2, SEQ):
        step(s_now, s_now % 2, storewait=True, gather=False)

    wait_store(0)
    wait_store(1)


@jax.jit
def _embed(idx_t, tp):
    run = functools.partial(
        pl.kernel,
        out_type=jax.ShapeDtypeStruct((SEQ, D, BATCH), jnp.float32),
        mesh=_mesh(),
        scratch_types=[
            pltpu.VMEM((SEQ, BBLK), jnp.int32),
            pltpu.VMEM((BBLK,), jnp.int32),
            pltpu.VMEM((BBLK,), jnp.int32),
            pltpu.VMEM((BBLK,), jnp.int32),
            pltpu.VMEM((BBLK,), jnp.int32),
            pltpu.VMEM((BBLK, 2 * D), jnp.float32),
            pltpu.VMEM((BBLK, 2 * D), jnp.float32),
            pltpu.VMEM((D, BBLK), jnp.float32),
            pltpu.VMEM((D, BBLK), jnp.float32),
            pltpu.SemaphoreType.DMA,
            pltpu.SemaphoreType.DMA,
            pltpu.SemaphoreType.DMA,
            pltpu.SemaphoreType.DMA,
        ],
        compiler_params=_SC_PARAMS,
    )(_gather_body)
    return run(tp, idx_t)


def kernel(input_, table):
    idx_t = input_.astype(jnp.int32).T
    tail_pairs = table[VOCAB - NTAIL:].reshape(NTAIL // 2, 2 * D)
    tp = _format_table(table.T, tail_pairs)
    out_t = _embed(idx_t, tp)
    return out_t.transpose(2, 0, 1)


# speedup vs baseline: 2.2837x; 2.2837x over previous
import functools
import math

import jax
import jax.numpy as jnp
from jax import lax
from jax.experimental import pallas as pl
from jax.experimental.pallas import tpu as pltpu
from jax.experimental.pallas import tpu_sc as plsc

D = 64
LANES = 16
NCORE = 2
NSUB = 16
NW = NCORE * NSUB

VOCAB = 1000000
BATCH = 4096
SEQ = 200
BBLK = BATCH // NW
VPAIR = VOCAB // 2

NFULL = VOCAB // 128
NTAIL = VOCAB - NFULL * 128
BLK_BASE = NFULL // NW
BLK_REM = NFULL - BLK_BASE * NW

SCALE = math.sqrt(D)

_SC_PARAMS = pltpu.CompilerParams(
    use_tc_tiling_on_sc=True, needs_layout_passes=False,
    disable_bounds_checks=True,
)


def _mesh():
    return plsc.VectorSubcoreMesh(
        core_axis_name="c", subcore_axis_name="s",
        num_cores=NCORE, num_subcores=NSUB,
    )


def _fmt_body(tt_hbm, tail_hbm, tp_hbm, s0, s1, d0, d1, gi0, gi1, go0, go1):
    sblk = (s0, s1)
    dblk = (d0, d1)
    isem = (gi0, gi1)
    osem = (go0, go1)

    c = lax.axis_index("c")
    s = lax.axis_index("s")
    wid = s * NCORE + c
    extra = jnp.minimum(wid, BLK_REM)
    start = wid * BLK_BASE + extra
    nblk = BLK_BASE + jnp.where(wid < BLK_REM, 1, 0)

    iota = lax.iota(jnp.int32, LANES)

    def start_in(i, b):
        vb = start + i
        pltpu.async_copy(
            tt_hbm.at[:, pl.ds(vb * 128, 128)], sblk[b], isem[b]
        )

    def wait_in(b):
        pltpu.make_async_copy(
            tt_hbm.at[:, pl.ds(0, 128)], sblk[b], isem[b]
        ).wait()

    def start_out(i, b):
        vb = start + i
        pltpu.async_copy(
            dblk[b], tp_hbm.at[pl.ds(vb * 64, 64)], osem[b]
        )

    def wait_out(b):
        pltpu.make_async_copy(
            dblk[b], tp_hbm.at[pl.ds(0, 64)], osem[b]
        ).wait()

    def transpose_block(b, nq):
        src = sblk[b]
        dst = dblk[b]

        @plsc.parallel_loop(0, nq, unroll=4)
        def _q(q):
            for k in range(8):
                h = k // 4
                rvec = iota + ((k * LANES) % D)
                cvec = jnp.broadcast_to(2 * q + h, (LANES,))
                vals = plsc.load_gather(src, [rvec, cvec])
                dst[q, pl.ds(k * LANES, LANES)] = vals

    start_in(0, 0)

    @pl.when(nblk > 1)
    def _():
        start_in(1, 1)

    wait_in(0)
    transpose_block(0, 64)
    start_out(0, 0)

    @pl.loop(1, nblk)
    def _blk(i):
        b = lax.rem(i, 2)

        @pl.when(b == 0)
        def _():
            @pl.when(i + 1 < nblk)
            def _():
                start_in(i + 1, 1)
            wait_in(0)
            wait_out(0)
            transpose_block(0, 64)
            start_out(i, 0)

        @pl.when(b == 1)
        def _():
            @pl.when(i + 1 < nblk)
            def _():
                start_in(i + 1, 0)
            wait_in(1)

            @pl.when(i > 2)
            def _():
                wait_out(1)

            transpose_block(1, 64)
            start_out(i, 1)

    wait_out(0)
    wait_out(1)

    @pl.when(wid == NW - 1)
    def _tail():
        pltpu.sync_copy(tail_hbm, dblk[0].at[pl.ds(0, NTAIL // 2)])
        pltpu.sync_copy(dblk[0].at[pl.ds(0, NTAIL // 2)],
                        tp_hbm.at[pl.ds(NFULL * 64, NTAIL // 2)])


@jax.jit
def _format_table(tt, tail_pairs):
    run = functools.partial(
        pl.kernel,
        out_type=jax.ShapeDtypeStruct((VPAIR, 2 * D), jnp.float32),
        mesh=_mesh(),
        scratch_types=[
            pltpu.VMEM((D, 128), jnp.float32),
            pltpu.VMEM((D, 128), jnp.float32),
            pltpu.VMEM((D, 2 * D), jnp.float32),
            pltpu.VMEM((D, 2 * D), jnp.float32),
            pltpu.SemaphoreType.DMA,
            pltpu.SemaphoreType.DMA,
            pltpu.SemaphoreType.DMA,
            pltpu.SemaphoreType.DMA,
        ],
        compiler_params=_SC_PARAMS,
    )(_fmt_body)
    return run(tt, tail_pairs)


def _gather_body(tp_hbm, idx_hbm, out_hbm,
                 idx_v, p0, p1, h0, h1, r0, r1, o0, o1,
                 g0, g1, q0, q1):
    pidx = (p0, p1)
    hoff = (h0, h1)
    rows = (r0, r1)
    outv = (o0, o1)
    gsem = (g0, g1)
    osem = (q0, q1)

    c = lax.axis_index("c")
    s = lax.axis_index("s")
    wid = s * NCORE + c
    bbase = wid * BBLK

    pltpu.sync_copy(idx_hbm.at[:, pl.ds(bbase, BBLK)], idx_v)

    iota = lax.iota(jnp.int32, LANES)

    def prep_indices(s2, b):
        for j in range(BBLK // LANES):
            sl = pl.ds(j * LANES, LANES)
            v = idx_v[s2, sl]
            pidx[b][sl] = lax.shift_right_logical(v, 1)
            hoff[b][sl] = (v & 1) * D

    def start_gather(b):
        pltpu.async_copy(tp_hbm.at[pidx[b]], rows[b], gsem[b])

    def wait_gather(b):
        pltpu.make_async_copy(tp_hbm.at[pidx[b]], rows[b], gsem[b]).wait()

    def start_store(s_now, b):
        pltpu.async_copy(
            outv[b], out_hbm.at[s_now, :, pl.ds(bbase, BBLK)], osem[b]
        )

    def wait_store(b):
        pltpu.make_async_copy(
            outv[b], out_hbm.at[0, :, pl.ds(bbase, BBLK)], osem[b]
        ).wait()

    def transpose_scale(b):
        src = rows[b]
        dst = outv[b]

        @plsc.parallel_loop(0, D, unroll=4)
        def _f(f):
            for j in range(BBLK // LANES):
                sl = pl.ds(j * LANES, LANES)
                rvec = iota + (j * LANES)
                cvec = hoff[b][sl] + f
                vals = plsc.load_gather(src, [rvec, cvec])
                dst[f, sl] = vals * SCALE

    def step(s_now, b, *, storewait, gather):
        wait_gather(b)
        if storewait:
            wait_store(b)
        transpose_scale(b)
        start_store(s_now, b)
        if gather:
            prep_indices(s_now + 2, b)
            start_gather(b)

    for b in range(2):
        prep_indices(b, b)
        start_gather(b)

    for s_now in range(2):
        step(s_now, s_now, storewait=False, gather=True)

    @pl.loop(1, SEQ // 2 - 1)
    def _main(i):
        for b in range(2):
            step(i * 2 + b, b, storewait=True, gather=True)

    for s_now in range(SEQ - 2, SEQ):
        step(s_now, s_now % 2, storewait=True, gather=False)

    wait_store(0)
    wait_store(1)


@jax.jit
def _embed(idx_t, tp):
    run = functools.partial(
        pl.kernel,
        out_type=jax.ShapeDtypeStruct((SEQ, D, BATCH), jnp.float32),
        mesh=_mesh(),
        scratch_types=[
            pltpu.VMEM((SEQ, BBLK), jnp.int32),
            pltpu.VMEM((BBLK,), jnp.int32),
            pltpu.VMEM((BBLK,), jnp.int32),
            pltpu.VMEM((BBLK,), jnp.int32),
            pltpu.VMEM((BBLK,), jnp.int32),
            pltpu.VMEM((BBLK, 2 * D), jnp.float32),
            pltpu.VMEM((BBLK, 2 * D), jnp.float32),
            pltpu.VMEM((D, BBLK), jnp.float32),
            pltpu.VMEM((D, BBLK), jnp.float32),
            pltpu.SemaphoreType.DMA,
            pltpu.SemaphoreType.DMA,
            pltpu.SemaphoreType.DMA,
            pltpu.SemaphoreType.DMA,
        ],
        compiler_params=_SC_PARAMS,
    )(_gather_body)
    return run(tp, idx_t)


def kernel(input_, table):
    idx_t = input_.astype(jnp.int32).T
    tail_pairs = table[VOCAB - NTAIL:].reshape(NTAIL // 2, 2 * D)
    tp = _format_table(table.T, tail_pairs)
    out_t = _embed(idx_t, tp)
    return out_t.transpose(2, 0, 1)
